# baseline (speedup 1.0000x reference)
import functools

import jax
import jax.numpy as jnp
from jax import lax
from jax.experimental import pallas as pl
from jax.experimental.pallas import tpu as pltpu
from jax.experimental.pallas import tpu_sc as plsc

T = 2048
D = 768
E = 8
DFF = 3072
TOPK = 2
ROWS = T * TOPK
BT = 256
NB = ROWS // BT + E - 1
NPAD = NB * BT
BF = 512
NF = DFF // BF


def _gate_body(x_ref, gw_ref, comb_ref, eidx_ref, xb_ref):
    x = x_ref[...]
    logits = jax.lax.dot_general(
        x, gw_ref[...], (((1,), (1,)), ((), ())),
        preferred_element_type=jnp.float32)
    idx = jax.lax.broadcasted_iota(jnp.int32, (T, E), 1)
    v1 = jnp.max(logits, axis=1, keepdims=True)
    s1 = jnp.min(jnp.where(logits == v1, idx, E), axis=1, keepdims=True)
    masked = jnp.where(idx == s1, jnp.float32(-1e30), logits)
    v2 = jnp.max(masked, axis=1, keepdims=True)
    s2 = jnp.min(jnp.where(masked == v2, idx, E), axis=1, keepdims=True)
    z = jnp.exp(v2 - v1)
    wa = 1.0 / (1.0 + z)
    wb = z * wa
    oh1 = (idx == s1).astype(jnp.float32)
    oh2 = (idx == s2).astype(jnp.float32)
    comb_ref[...] = wa * oh1 + wb * oh2
    eidx_ref[...] = jnp.concatenate([s1, s2], axis=1)
    xb_ref[...] = x.astype(jnp.bfloat16)


def _gate(x, gate_w):
    return pl.pallas_call(
        _gate_body,
        out_shape=[
            jax.ShapeDtypeStruct((T, E), jnp.float32),
            jax.ShapeDtypeStruct((T, TOPK), jnp.int32),
            jax.ShapeDtypeStruct((T, D), jnp.bfloat16),
        ],
    )(x, gate_w)


_L = 16


def _lane_scalar(vec, lane):
    io = lax.broadcasted_iota(jnp.int32, (_L,), 0)
    return jnp.sum(jnp.where(io == lane, vec, 0), axis=0)


def _bin_body(eflat_hbm, gidx_hbm, bexp_hbm, vcnt_hbm, cnts_hbm,
              ev_ref, full_ref, list_ref, hist_ref, histall_ref,
              stage_ref, zero_ref, hist_sh):
    cid = lax.axis_index("c")
    sid = lax.axis_index("s")
    io = lax.broadcasted_iota(jnp.int32, (_L,), 0)

    chunk = ROWS // _L
    base = sid * chunk
    pltpu.sync_copy(eflat_hbm.at[pl.ds(base, chunk)], ev_ref)

    def hist_step(v, hist):
        vec = ev_ref[pl.ds(v * _L, _L)]
        for e in range(E):
            pc = jnp.sum(jnp.where(vec == e, 1, 0), axis=0)
            hist = hist + jnp.where(io == e, pc, 0)
        return hist

    hist = jnp.zeros((_L,), jnp.int32)
    for v in range(chunk // _L):
        hist = hist_step(v, hist)
    hist_ref[...] = hist
    pltpu.sync_copy(hist_ref, hist_sh.at[sid + _L])
    plsc.subcore_barrier()
    pltpu.sync_copy(hist_sh.at[pl.ds(_L, _L)], histall_ref)
    counts = jnp.zeros((_L,), jnp.int32)
    for w in range(_L):
        counts = counts + histall_ref[w, :]
    pc = lax.shift_left(lax.shift_right_logical(counts + (BT - 1), 8), 8)
    starts = jnp.zeros((_L,), jnp.int32)
    for k in range(E - 1):
        starts = starts + jnp.where(io > k, _lane_scalar(pc, k), 0)
    nblk_tot = lax.shift_right_logical(jnp.sum(pc, axis=0), 8)

    on_core0 = cid == 0

    @pl.when(on_core0 & (sid < E))
    def _compact():
        e = sid
        pltpu.sync_copy(eflat_hbm, full_ref)
        for v in range((2048 + _L) // _L):
            list_ref[pl.ds(v * _L, _L)] = jnp.zeros((_L,), jnp.int32)

        def scan_step(v, runcnt):
            vec = full_ref[pl.ds(v * _L, _L)]
            msk = vec == e
            mi = jnp.where(msk, 1, 0)
            ranks = plsc.cumsum(mi) - 1 + runcnt
            toks = lax.shift_right_logical(io + (v * _L), 1)
            plsc.store_scatter(list_ref, [ranks], toks, mask=msk)
            return runcnt + jnp.sum(mi, axis=0)

        runcnt = jnp.zeros((), jnp.int32)
        for v in range(ROWS // _L):
            runcnt = scan_step(v, runcnt)

        c_e = _lane_scalar(counts, e)
        start_e = pl.multiple_of(_lane_scalar(starts, e), BT)
        nblk_e = lax.shift_right_logical(c_e + (BT - 1), 8)
        for k in range(E):
            @pl.when(k < nblk_e)
            def _wr():
                pltpu.sync_copy(
                    list_ref.at[pl.ds(k * BT, BT)],
                    gidx_hbm.at[pl.ds(start_e + k * BT, BT)])

    @pl.when(on_core0 & (sid == E))
    def _zero_tail():
        for v in range(BT // _L):
            zero_ref[pl.ds(v * _L, _L)] = jnp.zeros((_L,), jnp.int32)
        for j in range(ROWS // BT, NB):
            @pl.when(j >= nblk_tot)
            def _z():
                pltpu.sync_copy(zero_ref, gidx_hbm.at[pl.ds(j * BT, BT)])

    @pl.when(on_core0 & (sid == E + 1))
    def _tables():
        for j in range(2):
            bidx = io + j * _L
            bexp = jnp.zeros((_L,), jnp.int32)
            vcnt = jnp.zeros((_L,), jnp.int32)
            for e in range(E):
                c_e = _lane_scalar(counts, e)
                lo = lax.shift_right_logical(_lane_scalar(starts, e), 8)
                nblk_e = lax.shift_right_logical(c_e + (BT - 1), 8)
                bexp = jnp.where(bidx >= lo, e, bexp)
                within = (bidx >= lo) & (bidx < lo + nblk_e)
                vc = jnp.minimum(c_e - (bidx - lo) * BT, BT)
                vcnt = jnp.where(within, vc, vcnt)
            stage_ref[pl.ds(j * _L, _L)] = bexp
            stage_ref[pl.ds(32 + j * _L, _L)] = vcnt
        stage_ref[pl.ds(64, _L)] = counts
        pltpu.sync_copy(stage_ref.at[pl.ds(0, 32)], bexp_hbm)
        pltpu.sync_copy(stage_ref.at[pl.ds(32, 32)], vcnt_hbm)
        pltpu.sync_copy(stage_ref.at[pl.ds(64, _L)], cnts_hbm)


def _bin(eflat):
    mesh = plsc.VectorSubcoreMesh(core_axis_name="c", subcore_axis_name="s")
    f = pl.kernel(
        _bin_body,
        mesh=mesh,
        out_type=[
            jax.ShapeDtypeStruct((NPAD,), jnp.int32),
            jax.ShapeDtypeStruct((32,), jnp.int32),
            jax.ShapeDtypeStruct((32,), jnp.int32),
            jax.ShapeDtypeStruct((_L,), jnp.int32),
        ],
        scratch_types=[
            pltpu.VMEM((ROWS // _L,), jnp.int32),
            pltpu.VMEM((ROWS,), jnp.int32),
            pltpu.VMEM((2048 + _L,), jnp.int32),
            pltpu.VMEM((_L,), jnp.int32),
            pltpu.VMEM((_L, _L), jnp.int32),
            pltpu.VMEM((80,), jnp.int32),
            pltpu.VMEM((BT,), jnp.int32),
            pltpu.VMEM_SHARED((2 * _L, _L), jnp.int32),
        ],
        compiler_params=pltpu.CompilerParams(needs_layout_passes=False),
    )
    return f(eflat)


_GROWS = NPAD // 32
_GC = (96, 88)


def _gather_body(xb_hbm, gidx_hbm, xg_hbm, idx0_ref, idx1_ref, rows_ref, sem):
    cid = lax.axis_index("c")
    sid = lax.axis_index("s")
    wid = sid * 2 + cid
    base = pl.multiple_of(wid * _GROWS, 8)
    pltpu.sync_copy(gidx_hbm.at[pl.ds(base, _GC[0])], idx0_ref)
    pltpu.sync_copy(gidx_hbm.at[pl.ds(base + _GC[0], _GC[1])], idx1_ref)
    pltpu.async_copy(xb_hbm.at[idx0_ref], rows_ref.at[pl.ds(0, _GC[0])],
                     sem).wait()
    pltpu.async_copy(xb_hbm.at[idx1_ref], rows_ref.at[pl.ds(_GC[0], _GC[1])],
                     sem).wait()
    pltpu.sync_copy(rows_ref, xg_hbm.at[pl.ds(base, _GROWS)])


def _gather(xb, gidx):
    xb32 = jax.lax.bitcast_convert_type(
        xb.reshape(T, D // 2, 2), jnp.int32)
    mesh = plsc.VectorSubcoreMesh(core_axis_name="c", subcore_axis_name="s")
    f = pl.kernel(
        _gather_body,
        mesh=mesh,
        out_type=jax.ShapeDtypeStruct((NPAD, D // 2), jnp.int32),
        scratch_types=[
            pltpu.VMEM((_GC[0],), jnp.int32),
            pltpu.VMEM((_GC[1],), jnp.int32),
            pltpu.VMEM((_GROWS, D // 2), jnp.int32),
            pltpu.SemaphoreType.DMA,
        ],
        compiler_params=pltpu.CompilerParams(needs_layout_passes=False),
    )
    xg32 = f(xb32, gidx)
    return jax.lax.bitcast_convert_type(
        xg32, jnp.bfloat16).reshape(NPAD, D)


def _expert_body(bexp_sm, vcnt_sm, cnts_sm,
                 xg_ref, w1_ref, w2_ref, comb_ref, out_ref, hs_ref, m_ref):
    f = pl.program_id(0)
    b = pl.program_id(1)
    e_b = bexp_sm[b]
    vc = vcnt_sm[b]

    @pl.when(b == 0)
    def _zero_hs():
        hs_ref[...] = jnp.zeros_like(hs_ref)

    @pl.when((f == 0) & (b == 0))
    def _zero_m():
        m_ref[...] = jnp.zeros_like(m_ref)

    @pl.when(vc > 0)
    def _block():
        xgb = xg_ref[pl.ds(b * BT, BT), :]
        w1b = w1_ref[0].astype(jnp.bfloat16)
        h = jax.lax.dot_general(xgb, w1b, (((1,), (1,)), ((), ())),
                                preferred_element_type=jnp.float32)
        rid = jax.lax.broadcasted_iota(jnp.int32, (BT, 1), 0)
        h = jnp.where(rid < vc, h, 0.0)
        h = h * (1.0 / (1.0 + jnp.exp(-h)))
        valid = (rid < vc).astype(jnp.float32)
        sv = jax.lax.dot_general(valid, h, (((0,), (0,)), ((), ())),
                                 preferred_element_type=jnp.float32)
        scale = 1.0 / jnp.maximum(cnts_sm[e_b], 1).astype(jnp.float32)
        hs_ref[pl.ds(e_b, 1), :] += sv * scale

    @pl.when(b == NB - 1)
    def _w2():
        mp = jax.lax.dot_general(
            hs_ref[...], w2_ref[...], (((1,), (2,)), ((0,), (0,))),
            preferred_element_type=jnp.float32)
        m_ref[...] += mp

    @pl.when((f == NF - 1) & (b == NB - 1))
    def _combine():
        out_ref[...] = jax.lax.dot_general(
            comb_ref[...], m_ref[...], (((1,), (0,)), ((), ())),
            preferred_element_type=jnp.float32)


def _expert(bexp, vcnt, cnts, xg, W1, W2, comb):
    grid_spec = pltpu.PrefetchScalarGridSpec(
        num_scalar_prefetch=3,
        grid=(NF, NB),
        in_specs=[
            pl.BlockSpec((NPAD, D), lambda f, b, be, vn, cn: (0, 0)),
            pl.BlockSpec((1, BF, D), lambda f, b, be, vn, cn: (be[b], f, 0)),
            pl.BlockSpec((E, D, BF), lambda f, b, be, vn, cn: (0, 0, f)),
            pl.BlockSpec((T, E), lambda f, b, be, vn, cn: (0, 0)),
        ],
        out_specs=pl.BlockSpec((T, D), lambda f, b, be, vn, cn: (0, 0)),
        scratch_shapes=[
            pltpu.VMEM((E, BF), jnp.float32),
            pltpu.VMEM((E, D), jnp.float32),
        ],
    )
    return pl.pallas_call(
        _expert_body,
        grid_spec=grid_spec,
        out_shape=jax.ShapeDtypeStruct((T, D), jnp.float32),
    )(bexp, vcnt, cnts, xg, W1, W2, comb)


@jax.jit
def kernel(hidden_states, gate_w, W1, W2):
    b, s_len, d = hidden_states.shape
    x = hidden_states.reshape(T, D)
    comb, eidx, xb = _gate(x, gate_w)
    gidx, bexp, vcnt, cnts = _bin(eidx.reshape(ROWS))
    xg = _gather(xb, gidx)
    out = _expert(bexp, vcnt, cnts, xg, W1, W2, comb)
    return out.reshape(b, s_len, d)


# speedup vs baseline: 1.0003x; 1.0003x over previous
import functools

import jax
import jax.numpy as jnp
from jax import lax
from jax.experimental import pallas as pl
from jax.experimental.pallas import tpu as pltpu
from jax.experimental.pallas import tpu_sc as plsc

T = 2048
D = 768
E = 8
DFF = 3072
TOPK = 2
ROWS = T * TOPK
BT = 256
NB = ROWS // BT + E - 1
NPAD = NB * BT
BF = 512
NF = DFF // BF


def _gate_body(x_ref, gw_ref, comb_ref, eidx_ref, xb_ref):
    x = x_ref[...]
    logits = jax.lax.dot_general(
        x, gw_ref[...], (((1,), (1,)), ((), ())),
        preferred_element_type=jnp.float32)
    idx = jax.lax.broadcasted_iota(jnp.int32, (T, E), 1)
    v1 = jnp.max(logits, axis=1, keepdims=True)
    s1 = jnp.min(jnp.where(logits == v1, idx, E), axis=1, keepdims=True)
    masked = jnp.where(idx == s1, jnp.float32(-1e30), logits)
    v2 = jnp.max(masked, axis=1, keepdims=True)
    s2 = jnp.min(jnp.where(masked == v2, idx, E), axis=1, keepdims=True)
    z = jnp.exp(v2 - v1)
    wa = 1.0 / (1.0 + z)
    wb = z * wa
    oh1 = (idx == s1).astype(jnp.float32)
    oh2 = (idx == s2).astype(jnp.float32)
    comb_ref[...] = wa * oh1 + wb * oh2
    eidx_ref[...] = jnp.concatenate([s1, s2], axis=1)
    xb_ref[...] = x.astype(jnp.bfloat16)


def _gate(x, gate_w):
    return pl.pallas_call(
        _gate_body,
        out_shape=[
            jax.ShapeDtypeStruct((T, E), jnp.float32),
            jax.ShapeDtypeStruct((T, TOPK), jnp.int32),
            jax.ShapeDtypeStruct((T, D), jnp.bfloat16),
        ],
    )(x, gate_w)


_L = 16


def _lane_scalar(vec, lane):
    io = lax.broadcasted_iota(jnp.int32, (_L,), 0)
    return jnp.sum(jnp.where(io == lane, vec, 0), axis=0)


def _bin_body(eflat_hbm, gidx_hbm, bexp_hbm, vcnt_hbm, cnts_hbm,
              ev_ref, full_ref, list_ref, hist_ref, histall_ref,
              stage_ref, zero_ref, hist_sh):
    cid = lax.axis_index("c")
    sid = lax.axis_index("s")
    io = lax.broadcasted_iota(jnp.int32, (_L,), 0)

    chunk = ROWS // _L
    base = sid * chunk
    pltpu.sync_copy(eflat_hbm.at[pl.ds(base, chunk)], ev_ref)

    def hist_step(v, hist):
        vec = ev_ref[pl.ds(v * _L, _L)]
        for e in range(E):
            pc = jnp.sum(jnp.where(vec == e, 1, 0), axis=0)
            hist = hist + jnp.where(io == e, pc, 0)
        return hist

    hist = jnp.zeros((_L,), jnp.int32)
    for v in range(chunk // _L):
        hist = hist_step(v, hist)
    hist_ref[...] = hist
    pltpu.sync_copy(hist_ref, hist_sh.at[sid + _L])
    plsc.subcore_barrier()
    pltpu.sync_copy(hist_sh.at[pl.ds(_L, _L)], histall_ref)
    counts = jnp.zeros((_L,), jnp.int32)
    for w in range(_L):
        counts = counts + histall_ref[w, :]
    pc = lax.shift_left(lax.shift_right_logical(counts + (BT - 1), 8), 8)
    starts = jnp.zeros((_L,), jnp.int32)
    for k in range(E - 1):
        starts = starts + jnp.where(io > k, _lane_scalar(pc, k), 0)
    nblk_tot = lax.shift_right_logical(jnp.sum(pc, axis=0), 8)

    on_core0 = cid == 0

    @pl.when(on_core0 & (sid < E))
    def _compact():
        e = sid
        pltpu.sync_copy(eflat_hbm, full_ref)
        for v in range((2048 + _L) // _L):
            list_ref[pl.ds(v * _L, _L)] = jnp.zeros((_L,), jnp.int32)

        def scan_step(v, runcnt):
            vec = full_ref[pl.ds(v * _L, _L)]
            msk = vec == e
            mi = jnp.where(msk, 1, 0)
            ranks = plsc.cumsum(mi) - 1 + runcnt
            toks = lax.shift_right_logical(io + (v * _L), 1)
            plsc.store_scatter(list_ref, [ranks], toks, mask=msk)
            return runcnt + jnp.sum(mi, axis=0)

        runcnt = jnp.zeros((), jnp.int32)
        for v in range(ROWS // _L):
            runcnt = scan_step(v, runcnt)

        c_e = _lane_scalar(counts, e)
        start_e = pl.multiple_of(_lane_scalar(starts, e), BT)
        nblk_e = lax.shift_right_logical(c_e + (BT - 1), 8)
        for k in range(E):
            @pl.when(k < nblk_e)
            def _wr():
                pltpu.sync_copy(
                    list_ref.at[pl.ds(k * BT, BT)],
                    gidx_hbm.at[pl.ds(start_e + k * BT, BT)])

    @pl.when(on_core0 & (sid == E))
    def _zero_tail():
        for v in range(BT // _L):
            zero_ref[pl.ds(v * _L, _L)] = jnp.zeros((_L,), jnp.int32)
        for j in range(ROWS // BT, NB):
            @pl.when(j >= nblk_tot)
            def _z():
                pltpu.sync_copy(zero_ref, gidx_hbm.at[pl.ds(j * BT, BT)])

    @pl.when(on_core0 & (sid == E + 1))
    def _tables():
        for j in range(2):
            bidx = io + j * _L
            bexp = jnp.zeros((_L,), jnp.int32)
            vcnt = jnp.zeros((_L,), jnp.int32)
            for e in range(E):
                c_e = _lane_scalar(counts, e)
                lo = lax.shift_right_logical(_lane_scalar(starts, e), 8)
                nblk_e = lax.shift_right_logical(c_e + (BT - 1), 8)
                bexp = jnp.where(bidx >= lo, e, bexp)
                within = (bidx >= lo) & (bidx < lo + nblk_e)
                vc = jnp.minimum(c_e - (bidx - lo) * BT, BT)
                vcnt = jnp.where(within, vc, vcnt)
            stage_ref[pl.ds(j * _L, _L)] = bexp
            stage_ref[pl.ds(32 + j * _L, _L)] = vcnt
        stage_ref[pl.ds(64, _L)] = counts
        pltpu.sync_copy(stage_ref.at[pl.ds(0, 32)], bexp_hbm)
        pltpu.sync_copy(stage_ref.at[pl.ds(32, 32)], vcnt_hbm)
        pltpu.sync_copy(stage_ref.at[pl.ds(64, _L)], cnts_hbm)


def _bin(eflat):
    mesh = plsc.VectorSubcoreMesh(core_axis_name="c", subcore_axis_name="s")
    f = pl.kernel(
        _bin_body,
        mesh=mesh,
        out_type=[
            jax.ShapeDtypeStruct((NPAD,), jnp.int32),
            jax.ShapeDtypeStruct((32,), jnp.int32),
            jax.ShapeDtypeStruct((32,), jnp.int32),
            jax.ShapeDtypeStruct((_L,), jnp.int32),
        ],
        scratch_types=[
            pltpu.VMEM((ROWS // _L,), jnp.int32),
            pltpu.VMEM((ROWS,), jnp.int32),
            pltpu.VMEM((2048 + _L,), jnp.int32),
            pltpu.VMEM((_L,), jnp.int32),
            pltpu.VMEM((_L, _L), jnp.int32),
            pltpu.VMEM((80,), jnp.int32),
            pltpu.VMEM((BT,), jnp.int32),
            pltpu.VMEM_SHARED((2 * _L, _L), jnp.int32),
        ],
        compiler_params=pltpu.CompilerParams(needs_layout_passes=False),
    )
    return f(eflat)


_GROWS = NPAD // 32
_GC = (24, 24, 24, 24, 24, 24, 24, 16)


def _gather_body(xb_hbm, gidx_hbm, xg_hbm, *rest):
    idx_refs = rest[:len(_GC)]
    rows_ref, isem, gsem = rest[len(_GC):]
    cid = lax.axis_index("c")
    sid = lax.axis_index("s")
    wid = sid * 2 + cid
    base = pl.multiple_of(wid * _GROWS, 8)
    hs = []
    off = 0
    for i, n in enumerate(_GC):
        hs.append(pltpu.async_copy(
            gidx_hbm.at[pl.ds(base + off, n)], idx_refs[i], isem))
        off += n
    for h in hs:
        h.wait()
    hs = []
    off = 0
    for i, n in enumerate(_GC):
        hs.append(pltpu.async_copy(
            xb_hbm.at[idx_refs[i]], rows_ref.at[pl.ds(off, n)], gsem))
        off += n
    for h in hs:
        h.wait()
    pltpu.sync_copy(rows_ref, xg_hbm.at[pl.ds(base, _GROWS)])


def _gather(xb, gidx):
    xb32 = jax.lax.bitcast_convert_type(
        xb.reshape(T, D // 2, 2), jnp.int32)
    mesh = plsc.VectorSubcoreMesh(core_axis_name="c", subcore_axis_name="s")
    f = pl.kernel(
        _gather_body,
        mesh=mesh,
        out_type=jax.ShapeDtypeStruct((NPAD, D // 2), jnp.int32),
        scratch_types=(
            [pltpu.VMEM((n,), jnp.int32) for n in _GC] + [
                pltpu.VMEM((_GROWS, D // 2), jnp.int32),
                pltpu.SemaphoreType.DMA,
                pltpu.SemaphoreType.DMA,
            ]),
        compiler_params=pltpu.CompilerParams(needs_layout_passes=False),
    )
    xg32 = f(xb32, gidx)
    return jax.lax.bitcast_convert_type(
        xg32, jnp.bfloat16).reshape(NPAD, D)


def _expert_body(bexp_sm, vcnt_sm, cnts_sm,
                 xg_ref, w1_ref, w2_ref, comb_ref, out_ref, hs_ref, m_ref):
    f = pl.program_id(0)
    b = pl.program_id(1)
    e_b = bexp_sm[b]
    vc = vcnt_sm[b]

    @pl.when(b == 0)
    def _zero_hs():
        hs_ref[...] = jnp.zeros_like(hs_ref)

    @pl.when((f == 0) & (b == 0))
    def _zero_m():
        m_ref[...] = jnp.zeros_like(m_ref)

    @pl.when(vc > 0)
    def _block():
        xgb = xg_ref[pl.ds(b * BT, BT), :]
        w1b = w1_ref[0].astype(jnp.bfloat16)
        h = jax.lax.dot_general(xgb, w1b, (((1,), (1,)), ((), ())),
                                preferred_element_type=jnp.float32)
        rid = jax.lax.broadcasted_iota(jnp.int32, (BT, 1), 0)
        h = jnp.where(rid < vc, h, 0.0)
        h = h * (1.0 / (1.0 + jnp.exp(-h)))
        valid = (rid < vc).astype(jnp.float32)
        sv = jax.lax.dot_general(valid, h, (((0,), (0,)), ((), ())),
                                 preferred_element_type=jnp.float32)
        scale = 1.0 / jnp.maximum(cnts_sm[e_b], 1).astype(jnp.float32)
        hs_ref[pl.ds(e_b, 1), :] += sv * scale

    @pl.when(b == NB - 1)
    def _w2():
        mp = jax.lax.dot_general(
            hs_ref[...], w2_ref[...], (((1,), (2,)), ((0,), (0,))),
            preferred_element_type=jnp.float32)
        m_ref[...] += mp

    @pl.when((f == NF - 1) & (b == NB - 1))
    def _combine():
        out_ref[...] = jax.lax.dot_general(
            comb_ref[...], m_ref[...], (((1,), (0,)), ((), ())),
            preferred_element_type=jnp.float32)


def _expert(bexp, vcnt, cnts, xg, W1, W2, comb):
    grid_spec = pltpu.PrefetchScalarGridSpec(
        num_scalar_prefetch=3,
        grid=(NF, NB),
        in_specs=[
            pl.BlockSpec((NPAD, D), lambda f, b, be, vn, cn: (0, 0)),
            pl.BlockSpec((1, BF, D), lambda f, b, be, vn, cn: (be[b], f, 0)),
            pl.BlockSpec((E, D, BF), lambda f, b, be, vn, cn: (0, 0, f)),
            pl.BlockSpec((T, E), lambda f, b, be, vn, cn: (0, 0)),
        ],
        out_specs=pl.BlockSpec((T, D), lambda f, b, be, vn, cn: (0, 0)),
        scratch_shapes=[
            pltpu.VMEM((E, BF), jnp.float32),
            pltpu.VMEM((E, D), jnp.float32),
        ],
    )
    return pl.pallas_call(
        _expert_body,
        grid_spec=grid_spec,
        out_shape=jax.ShapeDtypeStruct((T, D), jnp.float32),
    )(bexp, vcnt, cnts, xg, W1, W2, comb)


@jax.jit
def kernel(hidden_states, gate_w, W1, W2):
    b, s_len, d = hidden_states.shape
    x = hidden_states.reshape(T, D)
    comb, eidx, xb = _gate(x, gate_w)
    gidx, bexp, vcnt, cnts = _bin(eidx.reshape(ROWS))
    xg = _gather(xb, gidx)
    out = _expert(bexp, vcnt, cnts, xg, W1, W2, comb)
    return out.reshape(b, s_len, d)


# speedup vs baseline: 2.5909x; 2.5901x over previous
import functools

import jax
import jax.numpy as jnp
from jax.experimental import pallas as pl
from jax.experimental.pallas import tpu as pltpu

_BF = 512


def _mega_body(x_ref, gw_ref, w1_ref, w2_ref, comb_ref, m_ref, mdc_ref, xb_ref):
    e = pl.program_id(0)
    f = pl.program_id(1)
    T, E = comb_ref.shape

    @pl.when((e == 0) & (f == 0))
    def _gate():
        x = x_ref[...]
        logits = jax.lax.dot_general(
            x, gw_ref[...], (((1,), (1,)), ((), ())),
            preferred_element_type=jnp.float32)
        idx = jax.lax.broadcasted_iota(jnp.int32, (T, E), 1)
        v1 = jnp.max(logits, axis=1, keepdims=True)
        s1 = jnp.min(jnp.where(logits == v1, idx, E), axis=1, keepdims=True)
        masked = jnp.where(idx == s1, jnp.float32(-1e30), logits)
        v2 = jnp.max(masked, axis=1, keepdims=True)
        s2 = jnp.min(jnp.where(masked == v2, idx, E), axis=1, keepdims=True)
        z = jnp.exp(v2 - v1)
        wa = 1.0 / (1.0 + z)
        wb = z * wa
        oh1 = (idx == s1).astype(jnp.float32)
        oh2 = (idx == s2).astype(jnp.float32)
        comb_ref[...] = wa * oh1 + wb * oh2
        mask = oh1 + oh2
        counts = jnp.sum(mask, axis=0, keepdims=True)
        mdc_ref[...] = mask / jnp.maximum(counts, 1.0)
        xb_ref[...] = x.astype(jnp.bfloat16)

    xb = xb_ref[...]
    w1e = w1_ref[0].astype(jnp.bfloat16)
    h = jax.lax.dot_general(xb, w1e, (((1,), (1,)), ((), ())),
                            preferred_element_type=jnp.float32)
    h = h * (1.0 / (1.0 + jnp.exp(-h)))
    onehot_e = (jax.lax.broadcasted_iota(jnp.int32, (1, E), 1) == e
                ).astype(jnp.float32)
    mcol = jax.lax.dot_general(mdc_ref[...], onehot_e, (((1,), (1,)), ((), ())),
                               preferred_element_type=jnp.float32)
    s = jax.lax.dot_general(mcol, h, (((0,), (0,)), ((), ())),
                            preferred_element_type=jnp.float32)
    part = jax.lax.dot_general(s, w2_ref[0], (((1,), (1,)), ((), ())),
                               preferred_element_type=jnp.float32)

    @pl.when(f == 0)
    def _init():
        m_ref[0] = part

    @pl.when(f != 0)
    def _acc():
        m_ref[0] = m_ref[0] + part


def _combine_body(comb_ref, m_ref, out_ref):
    out_ref[...] = jax.lax.dot_general(
        comb_ref[...], m_ref[...], (((1,), (0,)), ((), ())),
        preferred_element_type=jnp.float32)


@jax.jit
def kernel(hidden_states, gate_w, W1, W2):
    b, s_len, d = hidden_states.shape
    e_num, dff, _ = W1.shape
    t = b * s_len
    x = hidden_states.reshape(t, d)
    nf = dff // _BF

    comb, m = pl.pallas_call(
        _mega_body,
        grid=(e_num, nf),
        in_specs=[
            pl.BlockSpec((t, d), lambda e, f: (0, 0)),
            pl.BlockSpec((e_num, d), lambda e, f: (0, 0)),
            pl.BlockSpec((1, _BF, d), lambda e, f: (e, f, 0)),
            pl.BlockSpec((1, d, _BF), lambda e, f: (e, 0, f)),
        ],
        out_specs=[
            pl.BlockSpec((t, e_num), lambda e, f: (0, 0)),
            pl.BlockSpec((1, 1, d), lambda e, f: (e, 0, 0)),
        ],
        out_shape=[
            jax.ShapeDtypeStruct((t, e_num), jnp.float32),
            jax.ShapeDtypeStruct((e_num, 1, d), jnp.float32),
        ],
        scratch_shapes=[pltpu.VMEM((t, e_num), jnp.float32),
                        pltpu.VMEM((t, d), jnp.bfloat16)],
    )(x, gate_w, W1, W2)

    out = pl.pallas_call(
        _combine_body,
        out_shape=jax.ShapeDtypeStruct((t, d), jnp.float32),
    )(comb, m.reshape(e_num, d))
    return out.reshape(b, s_len, d)
